# baseline (speedup 1.0000x reference)
import functools

import jax
import jax.numpy as jnp
from jax import lax
from jax.experimental import pallas as pl
from jax.experimental.pallas import tpu as pltpu
from jax.experimental.pallas import tpu_sc as plsc

_DIM, _HID = 16, 64
_D = _DIM * _HID
_E = 64
_FF = 512
_T = 4096
_C = int(2.0 * _T / _E)

_B = 256
_NB = _T // _B

_NC, _NS = 2, 16
_NW = _NC * _NS
_TPW = _T // _NW
_CH = 64
_NCH = _TPW // _CH
_NROW = _T // _CH


def _route_body(x_ref, wg_ref, xbf_ref, dst_ref, gsrc_ref, gk_ref, aux_ref,
                acc_ref):
    i = pl.program_id(0)

    @pl.when(i == 0)
    def _():
        acc_ref[...] = jnp.zeros_like(acc_ref)

    xcat = x_ref[...]
    logits = lax.dot_general(xcat, wg_ref[...], (((1,), (0,)), ((), ())),
                             preferred_element_type=jnp.float32)

    def _rte(v):
        r = lax.bitcast_convert_type(v, jnp.int32)
        return r + (((r >> 16) & 1) + 0x7FFF)

    lo = _rte(xcat[:, :_D // 2])
    hi = _rte(xcat[:, _D // 2:])
    xbf_ref[...] = ((lo >> 16) & 0xFFFF) | (hi & jnp.int32(-65536))
    m = jnp.max(logits, axis=1, keepdims=True)
    ex = jnp.exp(logits - m)
    probs = ex / jnp.sum(ex, axis=1, keepdims=True)
    gate = jnp.max(probs, axis=1, keepdims=True)
    col = lax.broadcasted_iota(jnp.int32, (_B, _E), 1)
    eidx = jnp.min(jnp.where(probs == gate, col, _E), axis=1, keepdims=True)
    onehot = (col == eidx).astype(jnp.float32)

    r_i = lax.broadcasted_iota(jnp.int32, (_B, _B), 0)
    c_i = lax.broadcasted_iota(jnp.int32, (_B, _B), 1)
    tril = (c_i <= r_i).astype(jnp.float32)
    cs = lax.dot_general(tril, onehot, (((1,), (0,)), ((), ())),
                         preferred_element_type=jnp.float32)

    base = acc_ref[0:1, :]
    pos = (jnp.sum((cs + base) * onehot, axis=1, keepdims=True)
           - 1.0).astype(jnp.int32)
    keep = pos < _C
    gk = jnp.where(keep, gate, 0.0)
    dst = jnp.where(keep, eidx * _C + pos, _E * _C)
    gsrc = eidx * _C + jnp.minimum(pos, _C - 1)

    dst_ref[...] = dst
    gsrc_ref[...] = gsrc
    gk_ref[...] = gk

    acc_ref[0:1, :] = base + cs[_B - 1:_B, :]
    acc_ref[1:2, :] = acc_ref[1:2, :] + jnp.sum(probs, axis=0, keepdims=True)

    aux = jnp.sum(acc_ref[0:1, :] * acc_ref[1:2, :]) * (_E / (_T * _T))
    row0 = lax.broadcasted_iota(jnp.int32, (8, 128), 0) == 0
    col0 = lax.broadcasted_iota(jnp.int32, (8, 128), 1) == 0
    aux_ref[...] = jnp.where(row0 & col0, aux, 0.0)


def _route(x3, wg3):
    return pl.pallas_call(
        _route_body,
        grid=(_NB,),
        in_specs=[
            pl.BlockSpec((_B, _D), lambda i: (i, 0)),
            pl.BlockSpec((_D, _E), lambda i: (0, 0)),
        ],
        out_specs=[
            pl.BlockSpec((_B, _D // 2), lambda i: (i, 0)),
            pl.BlockSpec((_B, 1), lambda i: (i, 0)),
            pl.BlockSpec((_B, 1), lambda i: (i, 0)),
            pl.BlockSpec((_B, 1), lambda i: (i, 0)),
            pl.BlockSpec((8, 128), lambda i: (0, 0)),
        ],
        out_shape=[
            jax.ShapeDtypeStruct((_T, _D // 2), jnp.int32),
            jax.ShapeDtypeStruct((_T, 1), jnp.int32),
            jax.ShapeDtypeStruct((_T, 1), jnp.int32),
            jax.ShapeDtypeStruct((_T, 1), jnp.float32),
            jax.ShapeDtypeStruct((8, 128), jnp.float32),
        ],
        scratch_shapes=[pltpu.VMEM((8, _E), jnp.float32)],
        compiler_params=pltpu.CompilerParams(
            dimension_semantics=("arbitrary",)),
    )(x3, wg3)


_EPG = 2


def _ffn_body(buf_ref, w1_ref, b1_ref, w2_ref, b2_ref, y_ref):
    for ee in range(_EPG):
        w = buf_ref[ee * _C:(ee + 1) * _C]
        xlo = lax.bitcast_convert_type(w << 16, jnp.float32)
        xhi = lax.bitcast_convert_type(w & jnp.int32(-65536),
                                       jnp.float32)
        w1 = w1_ref[ee]
        h = (lax.dot_general(xlo, w1[:_D // 2], (((1,), (0,)), ((), ())),
                             preferred_element_type=jnp.float32)
             + lax.dot_general(xhi, w1[_D // 2:], (((1,), (0,)), ((), ())),
                               preferred_element_type=jnp.float32)
             + b1_ref[ee])
        h = jnp.maximum(h, 0.0)
        y = lax.dot_general(h, w2_ref[ee], (((1,), (0,)), ((), ())),
                            preferred_element_type=jnp.float32) + b2_ref[ee]

        def _rte(v):
            r = lax.bitcast_convert_type(v, jnp.int32)
            return r + (((r >> 16) & 1) + 0x7FFF)

        ylo = _rte(y[:, :_D // 2])
        yhi = _rte(y[:, _D // 2:])
        y_ref[ee * _C:(ee + 1) * _C] = (
            ((ylo >> 16) & 0xFFFF) | (yhi & jnp.int32(-65536)))


def _ffn(buf, w1, b1r, w2, b2r):
    return pl.pallas_call(
        _ffn_body,
        grid=(_E // _EPG,),
        in_specs=[
            pl.BlockSpec((_EPG * _C, _D // 2), lambda e: (e, 0)),
            pl.BlockSpec((_EPG, _D, _FF), lambda e: (e, 0, 0)),
            pl.BlockSpec((_EPG, 1, _FF), lambda e: (e, 0, 0)),
            pl.BlockSpec((_EPG, _FF, _D), lambda e: (e, 0, 0)),
            pl.BlockSpec((_EPG, 1, _D), lambda e: (e, 0, 0)),
        ],
        out_specs=pl.BlockSpec((_EPG * _C, _D // 2), lambda e: (e, 0)),
        out_shape=jax.ShapeDtypeStruct((_E * _C, _D // 2), jnp.int32),
        compiler_params=pltpu.CompilerParams(
            dimension_semantics=("arbitrary",)),
    )(buf, w1, b1r, w2, b2r)


_CHP = 32
_NP = _TPW // _CHP
_RING = 3


def _make_scatter():
    mesh = plsc.VectorSubcoreMesh(core_axis_name="c", subcore_axis_name="s")

    @functools.partial(
        pl.kernel,
        mesh=mesh,
        out_type=jax.ShapeDtypeStruct((_E * _C + _C, _D // 2), jnp.int32),
        scratch_types=[
            pltpu.VMEM((_RING, _CHP), jnp.int32),
            pltpu.VMEM((_RING, _CHP, _D // 2), jnp.int32),
            pltpu.SemaphoreType.DMA,
            pltpu.SemaphoreType.DMA,
            pltpu.SemaphoreType.DMA,
            pltpu.SemaphoreType.DMA,
            pltpu.SemaphoreType.DMA,
            pltpu.SemaphoreType.DMA,
        ],
    )
    def scatter_k(x_hbm, dst_hbm, buf_hbm, idx3, xb, g0, g1, g2, s0, s1, s2):
        gsem = (g0, g1, g2)
        ssem = (s0, s1, s2)
        w = lax.axis_index("s") * _NC + lax.axis_index("c")
        tok0 = w * _TPW
        stage, scat = {}, {}

        def issue_stage(k):
            stage[k] = pltpu.async_copy(
                x_hbm.at[pl.ds(tok0 + k * _CHP, _CHP)],
                xb.at[k % _RING], gsem[k % _RING])

        issue_stage(0)
        for k in range(_NP):
            if k + 1 < _NP:
                if k + 1 >= _RING:
                    scat[k + 1 - _RING].wait()
                issue_stage(k + 1)
            stage[k].wait()
            pltpu.sync_copy(dst_hbm.at[w * _NP + k], idx3.at[k % _RING])
            scat[k] = pltpu.async_copy(
                xb.at[k % _RING], buf_hbm.at[idx3.at[k % _RING]],
                ssem[k % _RING])
        for k in range(max(0, _NP - _RING), _NP):
            scat[k].wait()

    return scatter_k


def _make_combine():
    mesh = plsc.VectorSubcoreMesh(core_axis_name="c", subcore_axis_name="s")

    @functools.partial(
        pl.kernel,
        mesh=mesh,
        out_type=jax.ShapeDtypeStruct((_T, _D // 2), jnp.int32),
        scratch_types=[
            pltpu.VMEM((_TPW,), jnp.int32),
            pltpu.VMEM((_RING, _CHP, _D // 2), jnp.int32),
            pltpu.SemaphoreType.DMA,
            pltpu.SemaphoreType.DMA,
            pltpu.SemaphoreType.DMA,
            pltpu.SemaphoreType.DMA,
            pltpu.SemaphoreType.DMA,
        ],
    )
    def combine_k(y_hbm, gsrc_hbm, out_hbm,
                  idx_v, rows3, g0, g1, g2, s0, s1):
        gsem = (g0, g1, g2)
        ssem = (s0, s1)
        w = lax.axis_index("s") * _NC + lax.axis_index("c")
        tok0 = w * _TPW
        pltpu.sync_copy(gsrc_hbm.at[pl.ds(tok0, _TPW)], idx_v)
        gat, st = {}, {}

        def issue_gather(k):
            gat[k] = pltpu.async_copy(
                y_hbm.at[idx_v.at[pl.ds(k * _CHP, _CHP)]],
                rows3.at[k % _RING], gsem[k % _RING])

        issue_gather(0)
        for k in range(_NP):
            if k + 1 < _NP:
                if k + 1 >= _RING:
                    st[k + 1 - _RING].wait()
                issue_gather(k + 1)
            gat[k].wait()
            b = k % _RING
            st[k] = pltpu.async_copy(
                rows3.at[b], out_hbm.at[pl.ds(tok0 + k * _CHP, _CHP)],
                ssem[k % 2])
        for k in range(max(0, _NP - _RING), _NP):
            st[k].wait()

    return combine_k


_B2 = 512


def _finish_body(yg_ref, gk_ref, out_ref):
    g = pl.program_id(1)
    wv = yg_ref[...]
    s = gk_ref[...]
    half = jnp.where(g == 0, wv << 16, wv & jnp.int32(-65536))
    v = lax.bitcast_convert_type(half, jnp.float32) * s
    out_ref[...] = v.reshape(_B2, _DIM // 2, _HID)


def _finish(yg, gk):
    return pl.pallas_call(
        _finish_body,
        grid=(_T // _B2, 2),
        in_specs=[
            pl.BlockSpec((_B2, _D // 2), lambda i, g: (i, 0)),
            pl.BlockSpec((_B2, 1), lambda i, g: (i, 0)),
        ],
        out_specs=pl.BlockSpec((_B2, _DIM // 2, _HID), lambda i, g: (i, g, 0)),
        out_shape=jax.ShapeDtypeStruct((_T, _DIM, _HID), jnp.float32),
        compiler_params=pltpu.CompilerParams(
            dimension_semantics=("arbitrary", "arbitrary")),
    )(yg, gk)


def kernel(input_data, Wg, W1, b1, W2, b2):
    xbf, dst, gsrc, gk, aux3 = _route(input_data.reshape(_T, _D), Wg)
    dst2 = dst.reshape(_NW * _NP, _CHP)
    gsrc1 = gsrc.reshape(_T)

    buf = _make_scatter()(xbf, dst2)
    y_e = _ffn(buf, W1, b1.reshape(_E, 1, _FF), W2, b2.reshape(_E, 1, _D))
    yg = _make_combine()(y_e, gsrc1)
    out_moe = _finish(yg, gk)
    return out_moe, aux3[0, 0]


# speedup vs baseline: 1.1149x; 1.1149x over previous
import functools

import jax
import jax.numpy as jnp
from jax import lax
from jax.experimental import pallas as pl
from jax.experimental.pallas import tpu as pltpu
from jax.experimental.pallas import tpu_sc as plsc

_DIM, _HID = 16, 64
_D = _DIM * _HID
_E = 64
_FF = 512
_T = 4096
_C = int(2.0 * _T / _E)

_B = 256
_NB = _T // _B

_NC, _NS = 2, 16
_NW = _NC * _NS
_TPW = _T // _NW
_CH = 64
_NCH = _TPW // _CH
_NROW = _T // _CH


def _route_body(x_ref, wg_ref, xbf_ref, dst_ref, gsrc_ref, gk_ref, aux_ref,
                acc_ref):
    i = pl.program_id(0)

    @pl.when(i == 0)
    def _():
        acc_ref[...] = jnp.zeros_like(acc_ref)

    xcat = x_ref[...]
    logits = lax.dot_general(xcat, wg_ref[...], (((1,), (0,)), ((), ())),
                             preferred_element_type=jnp.float32)

    def _rte(v):
        r = lax.bitcast_convert_type(v, jnp.int32)
        return r + (((r >> 16) & 1) + 0x7FFF)

    lo = _rte(xcat[:, :_D // 2])
    hi = _rte(xcat[:, _D // 2:])
    xbf_ref[...] = ((lo >> 16) & 0xFFFF) | (hi & jnp.int32(-65536))
    m = jnp.max(logits, axis=1, keepdims=True)
    ex = jnp.exp(logits - m)
    probs = ex / jnp.sum(ex, axis=1, keepdims=True)
    gate = jnp.max(probs, axis=1, keepdims=True)
    col = lax.broadcasted_iota(jnp.int32, (_B, _E), 1)
    eidx = jnp.min(jnp.where(probs == gate, col, _E), axis=1, keepdims=True)
    onehot = (col == eidx).astype(jnp.float32)

    r_i = lax.broadcasted_iota(jnp.int32, (_B, _B), 0)
    c_i = lax.broadcasted_iota(jnp.int32, (_B, _B), 1)
    tril = (c_i <= r_i).astype(jnp.float32)
    cs = lax.dot_general(tril, onehot, (((1,), (0,)), ((), ())),
                         preferred_element_type=jnp.float32)

    base = acc_ref[0:1, :]
    pos = (jnp.sum((cs + base) * onehot, axis=1, keepdims=True)
           - 1.0).astype(jnp.int32)
    keep = pos < _C
    gk = jnp.where(keep, gate, 0.0)
    dst = jnp.where(keep, eidx * _C + pos, _E * _C)
    gsrc = eidx * _C + jnp.minimum(pos, _C - 1)

    dst_ref[...] = dst
    gsrc_ref[...] = gsrc
    gk_ref[...] = gk

    acc_ref[0:1, :] = base + cs[_B - 1:_B, :]
    acc_ref[1:2, :] = acc_ref[1:2, :] + jnp.sum(probs, axis=0, keepdims=True)

    aux = jnp.sum(acc_ref[0:1, :] * acc_ref[1:2, :]) * (_E / (_T * _T))
    row0 = lax.broadcasted_iota(jnp.int32, (8, 128), 0) == 0
    col0 = lax.broadcasted_iota(jnp.int32, (8, 128), 1) == 0
    aux_ref[...] = jnp.where(row0 & col0, aux, 0.0)


def _route(x3, wg3):
    return pl.pallas_call(
        _route_body,
        grid=(_NB,),
        in_specs=[
            pl.BlockSpec((_B, _D), lambda i: (i, 0)),
            pl.BlockSpec((_D, _E), lambda i: (0, 0)),
        ],
        out_specs=[
            pl.BlockSpec((_B, _D // 2), lambda i: (i, 0)),
            pl.BlockSpec((_B, 1), lambda i: (i, 0)),
            pl.BlockSpec((_B, 1), lambda i: (i, 0)),
            pl.BlockSpec((_B, 1), lambda i: (i, 0)),
            pl.BlockSpec((8, 128), lambda i: (0, 0)),
        ],
        out_shape=[
            jax.ShapeDtypeStruct((_T, _D // 2), jnp.int32),
            jax.ShapeDtypeStruct((_T, 1), jnp.int32),
            jax.ShapeDtypeStruct((_T, 1), jnp.int32),
            jax.ShapeDtypeStruct((_T, 1), jnp.float32),
            jax.ShapeDtypeStruct((8, 128), jnp.float32),
        ],
        scratch_shapes=[pltpu.VMEM((8, _E), jnp.float32)],
        compiler_params=pltpu.CompilerParams(
            dimension_semantics=("arbitrary",)),
    )(x3, wg3)


_EPG = 2


def _ffn_body(buf_ref, w1_ref, b1_ref, w2_ref, b2_ref, y_ref):
    for ee in range(_EPG):
        w = buf_ref[ee * _C:(ee + 1) * _C]
        xlo = lax.bitcast_convert_type(w << 16, jnp.float32)
        xhi = lax.bitcast_convert_type(w & jnp.int32(-65536),
                                       jnp.float32)
        w1 = w1_ref[ee]
        h = (lax.dot_general(xlo, w1[:_D // 2], (((1,), (0,)), ((), ())),
                             preferred_element_type=jnp.float32)
             + lax.dot_general(xhi, w1[_D // 2:], (((1,), (0,)), ((), ())),
                               preferred_element_type=jnp.float32)
             + b1_ref[ee])
        h = jnp.maximum(h, 0.0)
        y = lax.dot_general(h, w2_ref[ee], (((1,), (0,)), ((), ())),
                            preferred_element_type=jnp.float32) + b2_ref[ee]

        def _rte(v):
            r = lax.bitcast_convert_type(v, jnp.int32)
            return r + (((r >> 16) & 1) + 0x7FFF)

        ylo = _rte(y[:, :_D // 2])
        yhi = _rte(y[:, _D // 2:])
        y_ref[ee * _C:(ee + 1) * _C] = (
            ((ylo >> 16) & 0xFFFF) | (yhi & jnp.int32(-65536)))


def _ffn(buf, w1, b1r, w2, b2r):
    return pl.pallas_call(
        _ffn_body,
        grid=(_E // _EPG,),
        in_specs=[
            pl.BlockSpec((_EPG * _C, _D // 2), lambda e: (e, 0)),
            pl.BlockSpec((_EPG, _D, _FF), lambda e: (e, 0, 0)),
            pl.BlockSpec((_EPG, 1, _FF), lambda e: (e, 0, 0)),
            pl.BlockSpec((_EPG, _FF, _D), lambda e: (e, 0, 0)),
            pl.BlockSpec((_EPG, 1, _D), lambda e: (e, 0, 0)),
        ],
        out_specs=pl.BlockSpec((_EPG * _C, _D // 2), lambda e: (e, 0)),
        out_shape=jax.ShapeDtypeStruct((_E * _C, _D // 2), jnp.int32),
        compiler_params=pltpu.CompilerParams(
            dimension_semantics=("arbitrary",)),
    )(buf, w1, b1r, w2, b2r)


_CHP = 32
_NP = _TPW // _CHP
_RING = 3


def _make_scatter():
    mesh = plsc.VectorSubcoreMesh(core_axis_name="c", subcore_axis_name="s")

    @functools.partial(
        pl.kernel,
        mesh=mesh,
        out_type=jax.ShapeDtypeStruct((_E * _C + _C, _D // 2), jnp.int32),
        scratch_types=[
            pltpu.VMEM((_RING, _CHP), jnp.int32),
            pltpu.VMEM((_RING, _CHP, _D // 2), jnp.int32),
            pltpu.SemaphoreType.DMA,
            pltpu.SemaphoreType.DMA,
            pltpu.SemaphoreType.DMA,
            pltpu.SemaphoreType.DMA,
            pltpu.SemaphoreType.DMA,
            pltpu.SemaphoreType.DMA,
        ],
    )
    def scatter_k(x_hbm, dst_hbm, buf_hbm, idx3, xb, g0, g1, g2, s0, s1, s2):
        gsem = (g0, g1, g2)
        ssem = (s0, s1, s2)
        w = lax.axis_index("s") * _NC + lax.axis_index("c")
        tok0 = w * _TPW
        stage, scat = {}, {}

        def issue_stage(k):
            stage[k] = pltpu.async_copy(
                x_hbm.at[pl.ds(tok0 + k * _CHP, _CHP)],
                xb.at[k % _RING], gsem[k % _RING])

        issue_stage(0)
        for k in range(_NP):
            if k + 1 < _NP:
                if k + 1 >= _RING:
                    scat[k + 1 - _RING].wait()
                issue_stage(k + 1)
            stage[k].wait()
            pltpu.sync_copy(dst_hbm.at[w * _NP + k], idx3.at[k % _RING])
            scat[k] = pltpu.async_copy(
                xb.at[k % _RING], buf_hbm.at[idx3.at[k % _RING]],
                ssem[k % _RING])
        for k in range(max(0, _NP - _RING), _NP):
            scat[k].wait()

    return scatter_k


def _make_combine():
    mesh = plsc.VectorSubcoreMesh(core_axis_name="c", subcore_axis_name="s")

    @functools.partial(
        pl.kernel,
        mesh=mesh,
        out_type=jax.ShapeDtypeStruct((_T, _D // 2), jnp.int32),
        scratch_types=[
            pltpu.VMEM((_TPW,), jnp.int32),
            pltpu.VMEM((_RING, _CHP, _D // 2), jnp.int32),
            pltpu.SemaphoreType.DMA,
            pltpu.SemaphoreType.DMA,
            pltpu.SemaphoreType.DMA,
            pltpu.SemaphoreType.DMA,
            pltpu.SemaphoreType.DMA,
        ],
    )
    def combine_k(y_hbm, gsrc_hbm, out_hbm,
                  idx_v, rows3, g0, g1, g2, s0, s1):
        gsem = (g0, g1, g2)
        ssem = (s0, s1)
        w = lax.axis_index("s") * _NC + lax.axis_index("c")
        tok0 = w * _TPW
        pltpu.sync_copy(gsrc_hbm.at[pl.ds(tok0, _TPW)], idx_v)
        gat, st = {}, {}

        def issue_gather(k):
            gat[k] = pltpu.async_copy(
                y_hbm.at[idx_v.at[pl.ds(k * _CHP, _CHP)]],
                rows3.at[k % _RING], gsem[k % _RING])

        issue_gather(0)
        for k in range(_NP):
            if k + 1 < _NP:
                if k + 1 >= _RING:
                    st[k + 1 - _RING].wait()
                issue_gather(k + 1)
            gat[k].wait()
            b = k % _RING
            st[k] = pltpu.async_copy(
                rows3.at[b], out_hbm.at[pl.ds(tok0 + k * _CHP, _CHP)],
                ssem[k % 2])
        for k in range(max(0, _NP - _RING), _NP):
            st[k].wait()

    return combine_k


_B2 = 512


def _finish_body(yg_ref, gk_ref, out_ref):
    wv = yg_ref[...]
    s = gk_ref[...]
    lo = lax.bitcast_convert_type(wv << 16, jnp.float32) * s
    hi = lax.bitcast_convert_type(wv & jnp.int32(-65536), jnp.float32) * s
    out_ref[...] = jnp.concatenate([lo, hi], axis=1)


def _finish(yg, gk):
    return pl.pallas_call(
        _finish_body,
        grid=(_T // _B2,),
        in_specs=[
            pl.BlockSpec((_B2, _D // 2), lambda i: (i, 0)),
            pl.BlockSpec((_B2, 1), lambda i: (i, 0)),
        ],
        out_specs=pl.BlockSpec((_B2, _D), lambda i: (i, 0)),
        out_shape=jax.ShapeDtypeStruct((_T, _D), jnp.float32),
        compiler_params=pltpu.CompilerParams(
            dimension_semantics=("arbitrary",)),
    )(yg, gk)


def kernel(input_data, Wg, W1, b1, W2, b2):
    xbf, dst, gsrc, gk, aux3 = _route(input_data.reshape(_T, _D), Wg)
    dst2 = dst.reshape(_NW * _NP, _CHP)
    gsrc1 = gsrc.reshape(_T)

    buf = _make_scatter()(xbf, dst2)
    y_e = _ffn(buf, W1, b1.reshape(_E, 1, _FF), W2, b2.reshape(_E, 1, _D))
    yg = _make_combine()(y_e, gsrc1)
    out_moe = _finish(yg, gk).reshape(input_data.shape)
    return out_moe, aux3[0, 0]
